# baseline (speedup 1.0000x reference)
import jax
import jax.numpy as jnp
import numpy as np
from jax.experimental import pallas as pl

B, N, C = 4, 32768, 4
VOXEL_SIZE = np.array([0.05, 0.05, 0.1], np.float32)
BOUNDS = np.array([0.0, -40.0, -3.0, 70.4, 40.0, 1.0], np.float32)
MAX_VOXELS = 16000
MAX_OCC = 5
NUM_KEYPOINTS = 2048
GRID = np.round((BOUNDS[3:] - BOUNDS[:3]) / VOXEL_SIZE).astype(np.int32)
GX, GY, GZ = int(GRID[0]), int(GRID[1]), int(GRID[2])
SENTINEL = GX * GY * GZ


def _voxelize_one(pts):
    n = pts.shape[0]
    xyz = pts[:, :3]
    c = jnp.floor((xyz - jnp.asarray(BOUNDS[:3])) / jnp.asarray(VOXEL_SIZE)).astype(jnp.int32)
    valid = jnp.all((c >= 0) & (c < jnp.asarray(GRID)), axis=-1)
    key = c[:, 2] * (GY * GX) + c[:, 1] * GX + c[:, 0]
    key = jnp.where(valid, key, SENTINEL)
    uniq = jnp.unique(key, size=MAX_VOXELS, fill_value=SENTINEL)
    slot = jnp.searchsorted(uniq, key)
    slot_c = jnp.clip(slot, 0, MAX_VOXELS - 1)
    ok = (slot < MAX_VOXELS) & (uniq[slot_c] == key) & (key != SENTINEL)
    order = jnp.argsort(key)
    sk = key[order]
    start = jnp.searchsorted(sk, sk, side='left').astype(jnp.int32)
    rank = jnp.zeros((n,), jnp.int32).at[order].set(jnp.arange(n, dtype=jnp.int32) - start)
    ok = ok & (rank < MAX_OCC)
    slot_w = jnp.where(ok, slot_c, MAX_VOXELS)
    rank_w = jnp.where(ok, rank, 0)
    features = jnp.zeros((MAX_VOXELS, MAX_OCC, pts.shape[1]), pts.dtype).at[slot_w, rank_w].add(pts)
    occupancy = jnp.zeros((MAX_VOXELS,), jnp.int32).at[slot_w].add(1)
    z = uniq // (GY * GX)
    rem = uniq % (GY * GX)
    y = rem // GX
    x = rem % GX
    coords = jnp.stack([z, y, x], axis=1).astype(jnp.int32)
    coords = jnp.where((uniq != SENTINEL)[:, None], coords, -1)
    return features, coords, occupancy


def _fps(xyz, k):
    b, n, _ = xyz.shape

    def body(i, state):
        dist, idxs, last = state
        lp = jax.vmap(lambda p, j: p[j])(xyz, last)
        d = jnp.sum((xyz - lp[:, None, :]) ** 2, axis=-1)
        dist = jnp.minimum(dist, d)
        nxt = jnp.argmax(dist, axis=-1).astype(jnp.int32)
        idxs = idxs.at[:, i].set(nxt)
        return dist, idxs, nxt

    state = (jnp.full((b, n), 1e10, xyz.dtype), jnp.zeros((b, k), jnp.int32), jnp.zeros((b,), jnp.int32))
    _, idxs, _ = jax.lax.fori_loop(1, k, body, state)
    return idxs


def _identity_kernel(x_ref, o_ref):
    o_ref[...] = x_ref[...]


def kernel(points):
    flat = points.reshape(4096, 128)
    flat = pl.pallas_call(
        _identity_kernel,
        out_shape=jax.ShapeDtypeStruct(flat.shape, flat.dtype),
    )(flat)
    points = flat.reshape(points.shape)
    feats, coords, occs = [], [], []
    for i in range(points.shape[0]):
        f, c, o = _voxelize_one(points[i])
        c = jnp.concatenate([jnp.full((c.shape[0], 1), i, jnp.int32), c], axis=1)
        feats.append(f)
        coords.append(c)
        occs.append(o)
    features = jnp.concatenate(feats, axis=0)
    coordinates = jnp.concatenate(coords, axis=0)
    occupancy = jnp.concatenate(occs, axis=0)
    xyz = points[..., :3]
    idxs = _fps(jax.lax.stop_gradient(xyz), NUM_KEYPOINTS)
    keypoints = jax.vmap(lambda p, ii: p[ii])(xyz, idxs)
    return points, features, coordinates, occupancy, keypoints


# speedup vs baseline: 3.1344x; 3.1344x over previous
import jax
import jax.numpy as jnp
import numpy as np
from jax.experimental import pallas as pl
from jax.experimental.pallas import tpu as pltpu

B, N, C = 4, 32768, 4
VOXEL_SIZE = np.array([0.05, 0.05, 0.1], np.float32)
BOUNDS = np.array([0.0, -40.0, -3.0, 70.4, 40.0, 1.0], np.float32)
MAX_VOXELS = 16000
MAX_OCC = 5
NUM_KEYPOINTS = 2048
GRID = np.round((BOUNDS[3:] - BOUNDS[:3]) / VOXEL_SIZE).astype(np.int32)
GX, GY, GZ = int(GRID[0]), int(GRID[1]), int(GRID[2])
SENTINEL = GX * GY * GZ


def _voxelize_one(pts):
    n = pts.shape[0]
    xyz = pts[:, :3]
    c = jnp.floor((xyz - jnp.asarray(BOUNDS[:3])) / jnp.asarray(VOXEL_SIZE)).astype(jnp.int32)
    valid = jnp.all((c >= 0) & (c < jnp.asarray(GRID)), axis=-1)
    key = c[:, 2] * (GY * GX) + c[:, 1] * GX + c[:, 0]
    key = jnp.where(valid, key, SENTINEL)
    uniq = jnp.unique(key, size=MAX_VOXELS, fill_value=SENTINEL)
    slot = jnp.searchsorted(uniq, key)
    slot_c = jnp.clip(slot, 0, MAX_VOXELS - 1)
    ok = (slot < MAX_VOXELS) & (uniq[slot_c] == key) & (key != SENTINEL)
    order = jnp.argsort(key)
    sk = key[order]
    start = jnp.searchsorted(sk, sk, side='left').astype(jnp.int32)
    rank = jnp.zeros((n,), jnp.int32).at[order].set(jnp.arange(n, dtype=jnp.int32) - start)
    ok = ok & (rank < MAX_OCC)
    slot_w = jnp.where(ok, slot_c, MAX_VOXELS)
    rank_w = jnp.where(ok, rank, 0)
    features = jnp.zeros((MAX_VOXELS, MAX_OCC, pts.shape[1]), pts.dtype).at[slot_w, rank_w].add(pts)
    occupancy = jnp.zeros((MAX_VOXELS,), jnp.int32).at[slot_w].add(1)
    z = uniq // (GY * GX)
    rem = uniq % (GY * GX)
    y = rem // GX
    x = rem % GX
    coords = jnp.stack([z, y, x], axis=1).astype(jnp.int32)
    coords = jnp.where((uniq != SENTINEL)[:, None], coords, -1)
    return features, coords, occupancy


def _fps_body(xs_ref, ys_ref, zs_ref, out_ref, dist_ref):
    rows = xs_ref.shape[1]
    k = out_ref.shape[1]
    lane = jax.lax.broadcasted_iota(jnp.int32, (rows, 128), 1)
    row = jax.lax.broadcasted_iota(jnp.int32, (rows, 128), 0)
    lin = row * 128 + lane

    xs = xs_ref[0]
    ys = ys_ref[0]
    zs = zs_ref[0]

    px0 = xs[0, 0]
    py0 = ys[0, 0]
    pz0 = zs[0, 0]
    out_ref[0, 0, :] = jnp.stack([px0, py0, pz0])
    dist_ref[...] = jnp.full((rows, 128), 1e10, jnp.float32)

    def step(i, carry):
        px, py, pz = carry
        dx = xs - px
        dy = ys - py
        dz = zs - pz
        d = (dx * dx + dy * dy) + dz * dz
        dist = jnp.minimum(dist_ref[...], d)
        dist_ref[...] = dist
        m = jnp.max(dist)
        cand = jnp.where(dist == m, lin, jnp.int32(2**30))
        nxt = jnp.min(cand)
        r = nxt // 128
        c = nxt - r * 128
        lanemask = jax.lax.broadcasted_iota(jnp.int32, (1, 128), 1) == c
        nx = jnp.sum(jnp.where(lanemask, xs_ref[0, pl.ds(r, 1), :], 0.0))
        ny = jnp.sum(jnp.where(lanemask, ys_ref[0, pl.ds(r, 1), :], 0.0))
        nz = jnp.sum(jnp.where(lanemask, zs_ref[0, pl.ds(r, 1), :], 0.0))
        out_ref[0, pl.ds(i, 1), :] = jnp.stack([nx, ny, nz]).reshape(1, 3)
        return nx, ny, nz

    jax.lax.fori_loop(1, k, step, (px0, py0, pz0))


def _fps_keypoints(points):
    b = points.shape[0]
    n = points.shape[1]
    rows = n // 128
    k = NUM_KEYPOINTS
    xs = points[..., 0].reshape(b, rows, 128)
    ys = points[..., 1].reshape(b, rows, 128)
    zs = points[..., 2].reshape(b, rows, 128)
    return pl.pallas_call(
        _fps_body,
        grid=(b,),
        in_specs=[pl.BlockSpec((1, rows, 128), lambda i: (i, 0, 0))] * 3,
        out_specs=pl.BlockSpec((1, k, 3), lambda i: (i, 0, 0)),
        out_shape=jax.ShapeDtypeStruct((b, k, 3), jnp.float32),
        scratch_shapes=[pltpu.VMEM((rows, 128), jnp.float32)],
    )(xs, ys, zs)


def kernel(points):
    feats, coords, occs = [], [], []
    for i in range(points.shape[0]):
        f, c, o = _voxelize_one(points[i])
        c = jnp.concatenate([jnp.full((c.shape[0], 1), i, jnp.int32), c], axis=1)
        feats.append(f)
        coords.append(c)
        occs.append(o)
    features = jnp.concatenate(feats, axis=0)
    coordinates = jnp.concatenate(coords, axis=0)
    occupancy = jnp.concatenate(occs, axis=0)
    keypoints = _fps_keypoints(points)
    return points, features, coordinates, occupancy, keypoints


# speedup vs baseline: 14.9299x; 4.7633x over previous
import functools

import jax
import jax.numpy as jnp
import numpy as np
from jax import lax
from jax.experimental import pallas as pl
from jax.experimental.pallas import tpu as pltpu
from jax.experimental.pallas import tpu_sc as plsc

B, N, C = 4, 32768, 4
VOXEL_SIZE = np.array([0.05, 0.05, 0.1], np.float32)
BOUNDS = np.array([0.0, -40.0, -3.0, 70.4, 40.0, 1.0], np.float32)
MAX_VOXELS = 16000
MAX_OCC = 5
NUM_KEYPOINTS = 2048
GRID = np.round((BOUNDS[3:] - BOUNDS[:3]) / VOXEL_SIZE).astype(np.int32)
GX, GY, GZ = int(GRID[0]), int(GRID[1]), int(GRID[2])
SENT = GX * GY * GZ

ROWS = N // 128
NW = 16
CH = N // NW
NVEC = CH // 16
FD = MAX_VOXELS * MAX_OCC
FPAD = 327680
CPAD = 65536
OPAD = 16384


def _tc_body(xs_ref, ys_ref, zs_ref, kp_ref, key_ref, zc_ref, yc_ref, xc_ref,
             dist_ref):
    k = kp_ref.shape[1]

    xs = xs_ref[0]
    ys = ys_ref[0]
    zs = zs_ref[0]

    cx = jnp.floor((xs - BOUNDS[0]) / VOXEL_SIZE[0]).astype(jnp.int32)
    cy = jnp.floor((ys - BOUNDS[1]) / VOXEL_SIZE[1]).astype(jnp.int32)
    cz = jnp.floor((zs - BOUNDS[2]) / VOXEL_SIZE[2]).astype(jnp.int32)
    valid = ((cx >= 0) & (cx < GX) & (cy >= 0) & (cy < GY)
             & (cz >= 0) & (cz < GZ))
    key = cz * (GY * GX) + cy * GX + cx
    key_ref[0] = jnp.where(valid, key, SENT)
    zc_ref[0] = cz
    yc_ref[0] = cy
    xc_ref[0] = cx

    lane = lax.broadcasted_iota(jnp.int32, (ROWS, 128), 1)
    row = lax.broadcasted_iota(jnp.int32, (ROWS, 128), 0)
    lin = row * 128 + lane

    px0 = xs[0, 0]
    py0 = ys[0, 0]
    pz0 = zs[0, 0]
    kp_ref[0, 0, :] = jnp.stack([px0, py0, pz0])
    dist_ref[...] = jnp.full((ROWS, 128), 1e10, jnp.float32)

    def step(i, carry):
        px, py, pz = carry
        dx = xs - px
        dy = ys - py
        dz = zs - pz
        d = (dx * dx + dy * dy) + dz * dz
        dist = jnp.minimum(dist_ref[...], d)
        dist_ref[...] = dist
        m = jnp.max(dist)
        cand = jnp.where(dist == m, lin, jnp.int32(2**30))
        nxt = jnp.min(cand)
        r = nxt // 128
        c = nxt - r * 128
        lanemask = lax.broadcasted_iota(jnp.int32, (1, 128), 1) == c
        nx = jnp.sum(jnp.where(lanemask, xs_ref[0, pl.ds(r, 1), :], 0.0))
        ny = jnp.sum(jnp.where(lanemask, ys_ref[0, pl.ds(r, 1), :], 0.0))
        nz = jnp.sum(jnp.where(lanemask, zs_ref[0, pl.ds(r, 1), :], 0.0))
        kp_ref[0, pl.ds(i, 1), :] = jnp.stack([nx, ny, nz]).reshape(1, 3)
        return nx, ny, nz

    lax.fori_loop(1, k, step, (px0, py0, pz0))


def _tc_stage(points):
    xs = points[..., 0].reshape(B, ROWS, 128)
    ys = points[..., 1].reshape(B, ROWS, 128)
    zs = points[..., 2].reshape(B, ROWS, 128)
    kp, keyp, zcp, ycp, xcp = pl.pallas_call(
        _tc_body,
        grid=(B,),
        in_specs=[pl.BlockSpec((1, ROWS, 128), lambda i: (i, 0, 0))] * 3,
        out_specs=[
            pl.BlockSpec((1, NUM_KEYPOINTS, 3), lambda i: (i, 0, 0)),
            pl.BlockSpec((1, ROWS, 128), lambda i: (i, 0, 0)),
            pl.BlockSpec((1, ROWS, 128), lambda i: (i, 0, 0)),
            pl.BlockSpec((1, ROWS, 128), lambda i: (i, 0, 0)),
            pl.BlockSpec((1, ROWS, 128), lambda i: (i, 0, 0)),
        ],
        out_shape=[
            jax.ShapeDtypeStruct((B, NUM_KEYPOINTS, 3), jnp.float32),
            jax.ShapeDtypeStruct((B, ROWS, 128), jnp.int32),
            jax.ShapeDtypeStruct((B, ROWS, 128), jnp.int32),
            jax.ShapeDtypeStruct((B, ROWS, 128), jnp.int32),
            jax.ShapeDtypeStruct((B, ROWS, 128), jnp.int32),
        ],
        scratch_shapes=[pltpu.VMEM((ROWS, 128), jnp.float32)],
    )(xs, ys, zs)
    return kp, keyp, zcp, ycp, xcp


_MESH = plsc.VectorSubcoreMesh(core_axis_name="c", subcore_axis_name="s")


def _make_sc():
    i32 = jnp.int32
    f32 = jnp.float32

    @functools.partial(
        pl.kernel,
        mesh=_MESH,
        compiler_params=pltpu.CompilerParams(
            needs_layout_passes=False, use_tc_tiling_on_sc=False),
        out_type=[
            jax.ShapeDtypeStruct((B * FD * 4,), f32),
            jax.ShapeDtypeStruct((B * MAX_VOXELS * 4,), i32),
            jax.ShapeDtypeStruct((B * MAX_VOXELS,), i32),
        ],
        scratch_types=[
            pltpu.VMEM((CH,), i32),
            pltpu.VMEM((CH,), i32),
            pltpu.VMEM((CH,), i32),
            pltpu.VMEM((CH,), i32),
            pltpu.VMEM((CH,), i32),
            pltpu.VMEM((CH,), i32),
            pltpu.VMEM((512,), i32),
            pltpu.VMEM((512,), i32),
            pltpu.VMEM((NW * 512,), i32),
            pltpu.VMEM((CH,), i32),
            pltpu.VMEM((CH,), i32),
            pltpu.VMEM((CH,), i32),
            pltpu.VMEM((CH,), i32),
            pltpu.VMEM((CH,), i32),
            pltpu.VMEM((CH,), i32),
            pltpu.VMEM((CH,), i32),
            pltpu.VMEM((CH,), i32),
            pltpu.VMEM((CH,), i32),
            pltpu.VMEM((CH,), i32),
            pltpu.VMEM((CH,), i32),
            pltpu.VMEM((CH,), i32),
            pltpu.VMEM((CH,), f32),
            pltpu.VMEM((CH,), f32),
            pltpu.VMEM((CH,), f32),
            pltpu.VMEM((CH,), f32),
            pltpu.VMEM((CH,), i32),
            pltpu.VMEM((CH,), i32),
            pltpu.VMEM((CH,), i32),
            pltpu.VMEM((CH,), i32),
            pltpu.VMEM((CH,), f32),
            pltpu.VMEM((CH,), i32),
            pltpu.VMEM((CH,), i32),
            pltpu.VMEM((16,), i32),
            pltpu.VMEM((16,), i32),
            pltpu.VMEM((NW * 16,), i32),
            pltpu.VMEM((NW * 16,), i32),
            pltpu.VMEM((8,), i32),
            pltpu.VMEM_SHARED((N,), i32),
            pltpu.VMEM_SHARED((N,), i32),
            pltpu.VMEM_SHARED((N,), i32),
            pltpu.VMEM_SHARED((N,), i32),
            pltpu.VMEM_SHARED((NW * 512,), i32),
            pltpu.VMEM_SHARED((NW * 16,), i32),
            pltpu.VMEM_SHARED((NW * 16,), i32),
            pltpu.VMEM_SHARED((N,), i32),
            pltpu.VMEM_SHARED((N,), i32),
            pltpu.VMEM_SHARED((N,), i32),
            pltpu.VMEM_SHARED((FPAD,), f32),
            pltpu.VMEM_SHARED((CPAD,), i32),
            pltpu.VMEM_SHARED((OPAD,), i32),
        ],
    )
    def vox(keys_hbm, zch, ych, xch, pxh, pyh, pzh, pwh,
            feat_hbm, coords_hbm, occ_hbm,
            ck, ci, cpos, db, rankb, lastb, ctr, basebuf, hgrid,
            lcumb, lstb, fdb, odb, cdb, fi0, fi1, fi2, fi3, cj1, cj2, cj3,
            pxc, pyc, pzc, pwc, zcc, ycc, xcc, onesb, zf, zi, patb,
            nub, lsb, nuall, lsall, prevb,
            kA, iA, kB, iB, hist_sp, nu_sp, ls_sp, dstF, dstO, dstC,
            ftab, ctab, otab):
        cid = lax.axis_index("c")
        wid = lax.axis_index("s")
        iota = lax.iota(i32, 16)
        z16 = jnp.zeros((16,), i32)
        stripe = wid * CH

        def shift_up(v, s, fill):
            sh = jnp.take(v, jnp.maximum(iota - s, 0))
            return jnp.where(iota >= s, sh, fill)

        def shift_down(v, s, fill):
            sh = jnp.take(v, jnp.minimum(iota + s, 15))
            return jnp.where(iota + s <= 15, sh, fill)

        def rank_and_last(d):
            rank = z16
            later = z16
            for s in range(1, 16):
                rank = rank + jnp.where(shift_up(d, s, -1) == d, 1, 0)
                later = later + jnp.where(shift_down(d, s, -2) == d, 1, 0)
            return rank, later == 0

        def psum(v):
            p = v
            for s in (1, 2, 4, 8):
                p = p + shift_up(p, s, 0)
            return p

        def pmax(v):
            p = v
            for s in (1, 2, 4, 8):
                p = jnp.maximum(p, shift_up(p, s, jnp.int32(-(2**30))))
            return p

        def bcast_last(v):
            return jnp.take(v, z16 + 15)

        def do_batch(bi, _bcarry):
            b = cid * 2 + bi
            nbase = b * N

            def init_body(j, _):
                lpos = j * 16 + iota
                plsc.store_scatter(zf, [lpos], jnp.zeros((16,), f32))
                plsc.store_scatter(zi, [lpos], z16)
                plsc.store_scatter(onesb, [lpos], z16 + 1)
                plsc.store_scatter(patb, [lpos],
                                   jnp.where((lpos & 3) == 0, b, -1))
                plsc.store_scatter(ci, [lpos], stripe + lpos)
                return 0

            lax.fori_loop(0, NVEC, init_body, 0)

            pltpu.sync_copy(keys_hbm.at[pl.ds(nbase + stripe, CH)], ck)
            pltpu.sync_copy(ck, kA.at[pl.ds(stripe, CH)])
            pltpu.sync_copy(ci, iA.at[pl.ds(stripe, CH)])
            for j in range(FPAD // (NW * CH)):
                pltpu.sync_copy(
                    zf, ftab.at[pl.ds(wid * (FPAD // NW) + j * CH, CH)])
            for j in range(CPAD // (NW * CH)):
                pltpu.sync_copy(
                    patb, ctab.at[pl.ds(wid * (CPAD // NW) + j * CH, CH)])
            pltpu.sync_copy(zi.at[pl.ds(0, OPAD // NW)],
                            otab.at[pl.ds(wid * (OPAD // NW), OPAD // NW)])

            def radix_pass(p, _):
                shiftv = z16 + p * 9
                pltpu.sync_copy(kA.at[pl.ds(stripe, CH)], ck)
                pltpu.sync_copy(iA.at[pl.ds(stripe, CH)], ci)

                for j in range(512 // 16):
                    ctr[pl.ds(j * 16, 16)] = z16

                def hist_body(i, _h):
                    lpos = i * 16 + iota
                    k16 = plsc.load_gather(ck, [lpos])
                    d = jnp.right_shift(k16, shiftv) & 511
                    rank, is_last = rank_and_last(d)
                    plsc.store_scatter(db, [lpos], d)
                    plsc.store_scatter(rankb, [lpos], rank)
                    plsc.store_scatter(lastb, [lpos],
                                       jnp.where(is_last, 1, 0))
                    old = plsc.load_gather(ctr, [d])
                    plsc.store_scatter(ctr, [d], old + rank + 1,
                                       mask=is_last)
                    return 0

                lax.fori_loop(0, NVEC, hist_body, 0)
                pltpu.sync_copy(ctr, hist_sp.at[pl.ds(wid * 512, 512)])
                plsc.subcore_barrier()

                pltpu.sync_copy(hist_sp, hgrid)

                def base_body(cidx, carry):
                    d0 = cidx * 16
                    tot = z16
                    bef = z16
                    for t in range(NW):
                        h = plsc.load_gather(hgrid, [t * 512 + d0 + iota])
                        tot = tot + h
                        bef = bef + jnp.where(t < wid, h, 0)
                    incl = psum(tot)
                    base16 = carry + (incl - tot) + bef
                    plsc.store_scatter(basebuf, [d0 + iota], base16)
                    return carry + bcast_last(incl)

                lax.fori_loop(0, 512 // 16, base_body, z16)

                for j in range(512 // 16):
                    ctr[pl.ds(j * 16, 16)] = z16

                def scat_body(i, _s):
                    lpos = i * 16 + iota
                    d = plsc.load_gather(db, [lpos])
                    rank = plsc.load_gather(rankb, [lpos])
                    is_last = plsc.load_gather(lastb, [lpos]) == 1
                    cnt = plsc.load_gather(ctr, [d])
                    bs = plsc.load_gather(basebuf, [d])
                    plsc.store_scatter(cpos, [lpos], bs + cnt + rank)
                    plsc.store_scatter(ctr, [d], cnt + rank + 1,
                                       mask=is_last)
                    return 0

                lax.fori_loop(0, NVEC, scat_body, 0)
                pltpu.sync_copy(ck, kB.at[cpos])
                pltpu.sync_copy(ci, iB.at[cpos])
                plsc.subcore_barrier()
                pltpu.sync_copy(kB.at[pl.ds(stripe, CH)], ck)
                pltpu.sync_copy(ck, kA.at[pl.ds(stripe, CH)])
                pltpu.sync_copy(iB.at[pl.ds(stripe, CH)], ci)
                pltpu.sync_copy(ci, iA.at[pl.ds(stripe, CH)])
                plsc.subcore_barrier()
                return 0

            lax.fori_loop(0, 3, radix_pass, 0)

            @pl.when(wid > 0)
            def _():
                pltpu.sync_copy(kA.at[pl.ds(stripe - 8, 8)], prevb)

            pv = plsc.load_gather(prevb, [z16 + 7])
            pv = jnp.where(wid > 0, pv, -1)

            def c1_body(i, carry):
                runcum, prevk, runmax = carry
                lpos = i * 16 + iota
                pos = stripe + lpos
                k16 = plsc.load_gather(ck, [lpos])
                sh = shift_up(k16, 1, 0)
                sh = jnp.where(iota == 0, prevk, sh)
                isnew = jnp.where(k16 != sh, 1, 0)
                incl = psum(isnew)
                lcum16 = runcum + incl
                plsc.store_scatter(lcumb, [lpos], lcum16)
                s16 = jnp.where(isnew == 1, pos, -1)
                lst16 = jnp.maximum(runmax, pmax(s16))
                plsc.store_scatter(lstb, [lpos], lst16)
                return (bcast_last(lcum16), bcast_last(k16),
                        bcast_last(lst16))

            runcum, _pk, runmax = lax.fori_loop(
                0, NVEC, c1_body, (z16, pv, z16 - 1))
            nub[...] = runcum
            lsb[...] = runmax
            pltpu.sync_copy(nub, nu_sp.at[pl.ds(wid * 16, 16)])
            pltpu.sync_copy(lsb, ls_sp.at[pl.ds(wid * 16, 16)])
            plsc.subcore_barrier()

            pltpu.sync_copy(nu_sp, nuall)
            pltpu.sync_copy(ls_sp, lsall)
            pnu = plsc.load_gather(nuall, [iota * 16])
            pls = plsc.load_gather(lsall, [iota * 16])
            sb = bcast_last(psum(jnp.where(iota < wid, pnu, 0)))
            cin = bcast_last(pmax(jnp.where(iota < wid, pls, -1)))

            def c2_body(i, _c):
                lpos = i * 16 + iota
                pos = stripe + lpos
                k16 = plsc.load_gather(ck, [lpos])
                lcum16 = plsc.load_gather(lcumb, [lpos])
                lst16 = plsc.load_gather(lstb, [lpos])
                slot = sb + lcum16 - 1
                start = jnp.maximum(lst16, cin)
                rank = pos - start
                validk = k16 != SENT
                sok = slot < MAX_VOXELS
                ok = validk & sok & (rank < MAX_OCC)
                fdst = jnp.where(ok, slot * MAX_OCC + rank, FD)
                odst = jnp.where(ok, slot, MAX_VOXELS)
                cdst = jnp.where(validk & sok & (rank == 0), slot,
                                 MAX_VOXELS)
                plsc.store_scatter(fdb, [lpos], fdst)
                plsc.store_scatter(odb, [lpos], odst)
                plsc.store_scatter(cdb, [lpos], cdst)
                return 0

            lax.fori_loop(0, NVEC, c2_body, 0)
            pltpu.sync_copy(fdb, dstF.at[ci])
            pltpu.sync_copy(odb, dstO.at[ci])
            pltpu.sync_copy(cdb, dstC.at[ci])
            plsc.subcore_barrier()

            pltpu.sync_copy(dstF.at[pl.ds(stripe, CH)], fdb)
            pltpu.sync_copy(dstO.at[pl.ds(stripe, CH)], odb)
            pltpu.sync_copy(dstC.at[pl.ds(stripe, CH)], cdb)
            pltpu.sync_copy(pxh.at[pl.ds(nbase + stripe, CH)], pxc)
            pltpu.sync_copy(pyh.at[pl.ds(nbase + stripe, CH)], pyc)
            pltpu.sync_copy(pzh.at[pl.ds(nbase + stripe, CH)], pzc)
            pltpu.sync_copy(pwh.at[pl.ds(nbase + stripe, CH)], pwc)
            pltpu.sync_copy(zch.at[pl.ds(nbase + stripe, CH)], zcc)
            pltpu.sync_copy(ych.at[pl.ds(nbase + stripe, CH)], ycc)
            pltpu.sync_copy(xch.at[pl.ds(nbase + stripe, CH)], xcc)

            def d_body(i, _d):
                lpos = i * 16 + iota
                f16 = plsc.load_gather(fdb, [lpos])
                c16 = plsc.load_gather(cdb, [lpos])
                plsc.store_scatter(fi0, [lpos], f16 * 4)
                plsc.store_scatter(fi1, [lpos], f16 * 4 + 1)
                plsc.store_scatter(fi2, [lpos], f16 * 4 + 2)
                plsc.store_scatter(fi3, [lpos], f16 * 4 + 3)
                plsc.store_scatter(cj1, [lpos], c16 * 4 + 1)
                plsc.store_scatter(cj2, [lpos], c16 * 4 + 2)
                plsc.store_scatter(cj3, [lpos], c16 * 4 + 3)
                return 0

            lax.fori_loop(0, NVEC, d_body, 0)
            pltpu.sync_copy(pxc, ftab.at[fi0])
            pltpu.sync_copy(pyc, ftab.at[fi1])
            pltpu.sync_copy(pzc, ftab.at[fi2])
            pltpu.sync_copy(pwc, ftab.at[fi3])
            pltpu.sync_copy(zcc, ctab.at[cj1])
            pltpu.sync_copy(ycc, ctab.at[cj2])
            pltpu.sync_copy(xcc, ctab.at[cj3])
            pltpu.sync_copy(onesb, otab.at[odb], add=True)
            plsc.subcore_barrier()

            fs = FD * 4 // NW
            pltpu.sync_copy(ftab.at[pl.ds(wid * fs, fs)],
                            feat_hbm.at[pl.ds(b * FD * 4 + wid * fs, fs)])
            cs = MAX_VOXELS * 4 // NW
            pltpu.sync_copy(
                ctab.at[pl.ds(wid * cs, cs)],
                coords_hbm.at[pl.ds(b * MAX_VOXELS * 4 + wid * cs, cs)])
            osz = MAX_VOXELS // NW
            pltpu.sync_copy(
                otab.at[pl.ds(wid * osz, osz)],
                occ_hbm.at[pl.ds(b * MAX_VOXELS + wid * osz, osz)])
            plsc.subcore_barrier()
            return 0

        lax.fori_loop(0, B // 2, do_batch, 0)

    return vox


_SC_VOX = _make_sc()


def kernel(points):
    kp, keyp, zcp, ycp, xcp = _tc_stage(points)
    keys_flat = keyp.reshape(B * N)
    zc_flat = zcp.reshape(B * N)
    yc_flat = ycp.reshape(B * N)
    xc_flat = xcp.reshape(B * N)
    px = points[..., 0].reshape(B * N)
    py = points[..., 1].reshape(B * N)
    pz = points[..., 2].reshape(B * N)
    pw = points[..., 3].reshape(B * N)
    feat, coords, occ = _SC_VOX(keys_flat, zc_flat, yc_flat, xc_flat,
                                px, py, pz, pw)
    features = feat.reshape(B * MAX_VOXELS, MAX_OCC, 4)
    coordinates = coords.reshape(B * MAX_VOXELS, 4)
    occupancy = occ.reshape(B * MAX_VOXELS)
    return points, features, coordinates, occupancy, kp


# speedup vs baseline: 20.3806x; 1.3651x over previous
import functools

import jax
import jax.numpy as jnp
import numpy as np
from jax import lax
from jax.experimental import pallas as pl
from jax.experimental.pallas import tpu as pltpu
from jax.experimental.pallas import tpu_sc as plsc

B, N, C = 4, 32768, 4
VOXEL_SIZE = np.array([0.05, 0.05, 0.1], np.float32)
BOUNDS = np.array([0.0, -40.0, -3.0, 70.4, 40.0, 1.0], np.float32)
MAX_VOXELS = 16000
MAX_OCC = 5
NUM_KEYPOINTS = 2048
GRID = np.round((BOUNDS[3:] - BOUNDS[:3]) / VOXEL_SIZE).astype(np.int32)
GX, GY, GZ = int(GRID[0]), int(GRID[1]), int(GRID[2])
SENT = GX * GY * GZ

ROWS = N // 128
NW = 16
CH = N // NW
NVEC = CH // 16
FD = MAX_VOXELS * MAX_OCC
FPAD = 327680
CPAD = 65536
OPAD = 16384


def _keys_body(xs_ref, ys_ref, zs_ref, key_ref, zc_ref, yc_ref, xc_ref):
    xs = xs_ref[...]
    ys = ys_ref[...]
    zs = zs_ref[...]
    cx = jnp.floor((xs - BOUNDS[0]) / VOXEL_SIZE[0]).astype(jnp.int32)
    cy = jnp.floor((ys - BOUNDS[1]) / VOXEL_SIZE[1]).astype(jnp.int32)
    cz = jnp.floor((zs - BOUNDS[2]) / VOXEL_SIZE[2]).astype(jnp.int32)
    valid = ((cx >= 0) & (cx < GX) & (cy >= 0) & (cy < GY)
             & (cz >= 0) & (cz < GZ))
    key = cz * (GY * GX) + cy * GX + cx
    key_ref[...] = jnp.where(valid, key, SENT)
    zc_ref[...] = cz
    yc_ref[...] = cy
    xc_ref[...] = cx


def _keys_stage(xs, ys, zs):
    flat = (B * ROWS, 128)
    outs = pl.pallas_call(
        _keys_body,
        out_shape=[jax.ShapeDtypeStruct(flat, jnp.int32)] * 4,
    )(xs.reshape(flat), ys.reshape(flat), zs.reshape(flat))
    return outs


def _fps_body(xs_ref, ys_ref, zs_ref, kp_ref, dist_ref):
    k = kp_ref.shape[1]
    lane = lax.broadcasted_iota(jnp.int32, (ROWS, 128), 1)
    row = lax.broadcasted_iota(jnp.int32, (ROWS, 128), 0)
    lin = row * 128 + lane

    xsl = [xs_ref[b] for b in range(B)]
    ysl = [ys_ref[b] for b in range(B)]
    zsl = [zs_ref[b] for b in range(B)]

    p0 = []
    for b in range(B):
        px0 = xsl[b][0, 0]
        py0 = ysl[b][0, 0]
        pz0 = zsl[b][0, 0]
        kp_ref[b, 0, :] = jnp.stack([px0, py0, pz0])
        dist_ref[b] = jnp.full((ROWS, 128), 1e10, jnp.float32)
        p0 += [px0, py0, pz0]

    def step(i, carry):
        nxts = []
        for b in range(B):
            px, py, pz = carry[3 * b], carry[3 * b + 1], carry[3 * b + 2]
            dx = xsl[b] - px
            dy = ysl[b] - py
            dz = zsl[b] - pz
            d = (dx * dx + dy * dy) + dz * dz
            dist = jnp.minimum(dist_ref[b], d)
            dist_ref[b] = dist
            m = jnp.max(dist)
            cand = jnp.where(dist == m, lin, jnp.int32(2**30))
            nxt = jnp.min(cand)
            r = nxt // 128
            c = nxt - r * 128
            lanemask = lax.broadcasted_iota(jnp.int32, (1, 128), 1) == c
            nx = jnp.sum(jnp.where(lanemask, xs_ref[b, pl.ds(r, 1), :], 0.0))
            ny = jnp.sum(jnp.where(lanemask, ys_ref[b, pl.ds(r, 1), :], 0.0))
            nz = jnp.sum(jnp.where(lanemask, zs_ref[b, pl.ds(r, 1), :], 0.0))
            kp_ref[b, pl.ds(i, 1), :] = jnp.stack([nx, ny, nz]).reshape(1, 3)
            nxts += [nx, ny, nz]
        return tuple(nxts)

    lax.fori_loop(1, k, step, tuple(p0))


def _fps_stage(xs, ys, zs):
    return pl.pallas_call(
        _fps_body,
        out_shape=jax.ShapeDtypeStruct((B, NUM_KEYPOINTS, 3), jnp.float32),
        scratch_shapes=[pltpu.VMEM((B, ROWS, 128), jnp.float32)],
    )(xs, ys, zs)


_MESH = plsc.VectorSubcoreMesh(core_axis_name="c", subcore_axis_name="s")


def _make_sc():
    i32 = jnp.int32
    f32 = jnp.float32

    @functools.partial(
        pl.kernel,
        mesh=_MESH,
        compiler_params=pltpu.CompilerParams(
            needs_layout_passes=False, use_tc_tiling_on_sc=False),
        out_type=[
            jax.ShapeDtypeStruct((B * FD * 4,), f32),
            jax.ShapeDtypeStruct((B * MAX_VOXELS * 4,), i32),
            jax.ShapeDtypeStruct((B * MAX_VOXELS,), i32),
        ],
        scratch_types=[
            pltpu.VMEM((CH,), i32),
            pltpu.VMEM((CH,), i32),
            pltpu.VMEM((CH,), i32),
            pltpu.VMEM((CH,), i32),
            pltpu.VMEM((CH,), i32),
            pltpu.VMEM((CH,), i32),
            pltpu.VMEM((512,), i32),
            pltpu.VMEM((512,), i32),
            pltpu.VMEM((NW * 512,), i32),
            pltpu.VMEM((CH,), i32),
            pltpu.VMEM((CH,), i32),
            pltpu.VMEM((CH,), i32),
            pltpu.VMEM((CH,), i32),
            pltpu.VMEM((CH,), i32),
            pltpu.VMEM((CH,), i32),
            pltpu.VMEM((CH,), i32),
            pltpu.VMEM((CH,), i32),
            pltpu.VMEM((CH,), i32),
            pltpu.VMEM((CH,), i32),
            pltpu.VMEM((CH,), i32),
            pltpu.VMEM((CH,), i32),
            pltpu.VMEM((CH,), f32),
            pltpu.VMEM((CH,), f32),
            pltpu.VMEM((CH,), f32),
            pltpu.VMEM((CH,), f32),
            pltpu.VMEM((CH,), i32),
            pltpu.VMEM((CH,), i32),
            pltpu.VMEM((CH,), i32),
            pltpu.VMEM((CH,), i32),
            pltpu.VMEM((CH,), f32),
            pltpu.VMEM((CH,), i32),
            pltpu.VMEM((CH,), i32),
            pltpu.VMEM((16,), i32),
            pltpu.VMEM((16,), i32),
            pltpu.VMEM((NW * 16,), i32),
            pltpu.VMEM((NW * 16,), i32),
            pltpu.VMEM((8,), i32),
            pltpu.VMEM_SHARED((N,), i32),
            pltpu.VMEM_SHARED((N,), i32),
            pltpu.VMEM_SHARED((N,), i32),
            pltpu.VMEM_SHARED((N,), i32),
            pltpu.VMEM_SHARED((NW * 512,), i32),
            pltpu.VMEM_SHARED((NW * 16,), i32),
            pltpu.VMEM_SHARED((NW * 16,), i32),
            pltpu.VMEM_SHARED((N,), i32),
            pltpu.VMEM_SHARED((N,), i32),
            pltpu.VMEM_SHARED((N,), i32),
            pltpu.VMEM_SHARED((FPAD,), f32),
            pltpu.VMEM_SHARED((CPAD,), i32),
            pltpu.VMEM_SHARED((OPAD,), i32),
        ],
    )
    def vox(keys_hbm, zch, ych, xch, pxh, pyh, pzh, pwh,
            feat_hbm, coords_hbm, occ_hbm,
            ck, ci, cpos, db, rankb, lastb, ctr, basebuf, hgrid,
            lcumb, lstb, fdb, odb, cdb, fi0, fi1, fi2, fi3, cj1, cj2, cj3,
            pxc, pyc, pzc, pwc, zcc, ycc, xcc, onesb, zf, zi, patb,
            nub, lsb, nuall, lsall, prevb,
            kA, iA, kB, iB, hist_sp, nu_sp, ls_sp, dstF, dstO, dstC,
            ftab, ctab, otab):
        cid = lax.axis_index("c")
        wid = lax.axis_index("s")
        iota = lax.iota(i32, 16)
        z16 = jnp.zeros((16,), i32)
        stripe = wid * CH

        def shift_up(v, s, fill):
            sh = jnp.take(v, jnp.maximum(iota - s, 0))
            return jnp.where(iota >= s, sh, fill)

        def shift_down(v, s, fill):
            sh = jnp.take(v, jnp.minimum(iota + s, 15))
            return jnp.where(iota + s <= 15, sh, fill)

        def rank_and_last(d):
            rank = z16
            later = z16
            for s in range(1, 16):
                rank = rank + jnp.where(shift_up(d, s, -1) == d, 1, 0)
                later = later + jnp.where(shift_down(d, s, -2) == d, 1, 0)
            return rank, later == 0

        def psum(v):
            p = v
            for s in (1, 2, 4, 8):
                p = p + shift_up(p, s, 0)
            return p

        def pmax(v):
            p = v
            for s in (1, 2, 4, 8):
                p = jnp.maximum(p, shift_up(p, s, jnp.int32(-(2**30))))
            return p

        def bcast_last(v):
            return jnp.take(v, z16 + 15)

        def do_batch(bi, _bcarry):
            b = cid * 2 + bi
            nbase = b * N

            def init_body(j, _):
                lpos = j * 16 + iota
                plsc.store_scatter(zf, [lpos], jnp.zeros((16,), f32))
                plsc.store_scatter(zi, [lpos], z16)
                plsc.store_scatter(onesb, [lpos], z16 + 1)
                plsc.store_scatter(patb, [lpos],
                                   jnp.where((lpos & 3) == 0, b, -1))
                plsc.store_scatter(ci, [lpos], stripe + lpos)
                return 0

            lax.fori_loop(0, NVEC, init_body, 0)

            pltpu.sync_copy(keys_hbm.at[pl.ds(nbase + stripe, CH)], ck)
            pltpu.sync_copy(ck, kA.at[pl.ds(stripe, CH)])
            pltpu.sync_copy(ci, iA.at[pl.ds(stripe, CH)])
            for j in range(FPAD // (NW * CH)):
                pltpu.sync_copy(
                    zf, ftab.at[pl.ds(wid * (FPAD // NW) + j * CH, CH)])
            for j in range(CPAD // (NW * CH)):
                pltpu.sync_copy(
                    patb, ctab.at[pl.ds(wid * (CPAD // NW) + j * CH, CH)])
            pltpu.sync_copy(zi.at[pl.ds(0, OPAD // NW)],
                            otab.at[pl.ds(wid * (OPAD // NW), OPAD // NW)])

            def radix_pass(p, _):
                shiftv = z16 + p * 9
                pltpu.sync_copy(kA.at[pl.ds(stripe, CH)], ck)
                pltpu.sync_copy(iA.at[pl.ds(stripe, CH)], ci)

                for j in range(512 // 16):
                    ctr[pl.ds(j * 16, 16)] = z16

                def hist_body(i, _h):
                    lpos = i * 16 + iota
                    k16 = plsc.load_gather(ck, [lpos])
                    d = jnp.right_shift(k16, shiftv) & 511
                    rank, is_last = rank_and_last(d)
                    plsc.store_scatter(db, [lpos], d)
                    plsc.store_scatter(rankb, [lpos], rank)
                    plsc.store_scatter(lastb, [lpos],
                                       jnp.where(is_last, 1, 0))
                    old = plsc.load_gather(ctr, [d])
                    plsc.store_scatter(ctr, [d], old + rank + 1,
                                       mask=is_last)
                    return 0

                lax.fori_loop(0, NVEC, hist_body, 0)
                pltpu.sync_copy(ctr, hist_sp.at[pl.ds(wid * 512, 512)])
                plsc.subcore_barrier()

                pltpu.sync_copy(hist_sp, hgrid)

                def base_body(cidx, carry):
                    d0 = cidx * 16
                    tot = z16
                    bef = z16
                    for t in range(NW):
                        h = plsc.load_gather(hgrid, [t * 512 + d0 + iota])
                        tot = tot + h
                        bef = bef + jnp.where(t < wid, h, 0)
                    incl = psum(tot)
                    base16 = carry + (incl - tot) + bef
                    plsc.store_scatter(basebuf, [d0 + iota], base16)
                    return carry + bcast_last(incl)

                lax.fori_loop(0, 512 // 16, base_body, z16)

                for j in range(512 // 16):
                    ctr[pl.ds(j * 16, 16)] = z16

                def scat_body(i, _s):
                    lpos = i * 16 + iota
                    d = plsc.load_gather(db, [lpos])
                    rank = plsc.load_gather(rankb, [lpos])
                    is_last = plsc.load_gather(lastb, [lpos]) == 1
                    cnt = plsc.load_gather(ctr, [d])
                    bs = plsc.load_gather(basebuf, [d])
                    plsc.store_scatter(cpos, [lpos], bs + cnt + rank)
                    plsc.store_scatter(ctr, [d], cnt + rank + 1,
                                       mask=is_last)
                    return 0

                lax.fori_loop(0, NVEC, scat_body, 0)
                pltpu.sync_copy(ck, kB.at[cpos])
                pltpu.sync_copy(ci, iB.at[cpos])
                plsc.subcore_barrier()
                pltpu.sync_copy(kB.at[pl.ds(stripe, CH)], ck)
                pltpu.sync_copy(ck, kA.at[pl.ds(stripe, CH)])
                pltpu.sync_copy(iB.at[pl.ds(stripe, CH)], ci)
                pltpu.sync_copy(ci, iA.at[pl.ds(stripe, CH)])
                plsc.subcore_barrier()
                return 0

            lax.fori_loop(0, 3, radix_pass, 0)

            @pl.when(wid > 0)
            def _():
                pltpu.sync_copy(kA.at[pl.ds(stripe - 8, 8)], prevb)

            pv = plsc.load_gather(prevb, [z16 + 7])
            pv = jnp.where(wid > 0, pv, -1)

            def c1_body(i, carry):
                runcum, prevk, runmax = carry
                lpos = i * 16 + iota
                pos = stripe + lpos
                k16 = plsc.load_gather(ck, [lpos])
                sh = shift_up(k16, 1, 0)
                sh = jnp.where(iota == 0, prevk, sh)
                isnew = jnp.where(k16 != sh, 1, 0)
                incl = psum(isnew)
                lcum16 = runcum + incl
                plsc.store_scatter(lcumb, [lpos], lcum16)
                s16 = jnp.where(isnew == 1, pos, -1)
                lst16 = jnp.maximum(runmax, pmax(s16))
                plsc.store_scatter(lstb, [lpos], lst16)
                return (bcast_last(lcum16), bcast_last(k16),
                        bcast_last(lst16))

            runcum, _pk, runmax = lax.fori_loop(
                0, NVEC, c1_body, (z16, pv, z16 - 1))
            nub[...] = runcum
            lsb[...] = runmax
            pltpu.sync_copy(nub, nu_sp.at[pl.ds(wid * 16, 16)])
            pltpu.sync_copy(lsb, ls_sp.at[pl.ds(wid * 16, 16)])
            plsc.subcore_barrier()

            pltpu.sync_copy(nu_sp, nuall)
            pltpu.sync_copy(ls_sp, lsall)
            pnu = plsc.load_gather(nuall, [iota * 16])
            pls = plsc.load_gather(lsall, [iota * 16])
            sb = bcast_last(psum(jnp.where(iota < wid, pnu, 0)))
            cin = bcast_last(pmax(jnp.where(iota < wid, pls, -1)))

            def c2_body(i, _c):
                lpos = i * 16 + iota
                pos = stripe + lpos
                k16 = plsc.load_gather(ck, [lpos])
                lcum16 = plsc.load_gather(lcumb, [lpos])
                lst16 = plsc.load_gather(lstb, [lpos])
                slot = sb + lcum16 - 1
                start = jnp.maximum(lst16, cin)
                rank = pos - start
                validk = k16 != SENT
                sok = slot < MAX_VOXELS
                ok = validk & sok & (rank < MAX_OCC)
                fdst = jnp.where(ok, slot * MAX_OCC + rank, FD)
                odst = jnp.where(ok, slot, MAX_VOXELS)
                cdst = jnp.where(validk & sok & (rank == 0), slot,
                                 MAX_VOXELS)
                plsc.store_scatter(fdb, [lpos], fdst)
                plsc.store_scatter(odb, [lpos], odst)
                plsc.store_scatter(cdb, [lpos], cdst)
                return 0

            lax.fori_loop(0, NVEC, c2_body, 0)
            pltpu.sync_copy(fdb, dstF.at[ci])
            pltpu.sync_copy(odb, dstO.at[ci])
            pltpu.sync_copy(cdb, dstC.at[ci])
            plsc.subcore_barrier()

            pltpu.sync_copy(dstF.at[pl.ds(stripe, CH)], fdb)
            pltpu.sync_copy(dstO.at[pl.ds(stripe, CH)], odb)
            pltpu.sync_copy(dstC.at[pl.ds(stripe, CH)], cdb)
            pltpu.sync_copy(pxh.at[pl.ds(nbase + stripe, CH)], pxc)
            pltpu.sync_copy(pyh.at[pl.ds(nbase + stripe, CH)], pyc)
            pltpu.sync_copy(pzh.at[pl.ds(nbase + stripe, CH)], pzc)
            pltpu.sync_copy(pwh.at[pl.ds(nbase + stripe, CH)], pwc)
            pltpu.sync_copy(zch.at[pl.ds(nbase + stripe, CH)], zcc)
            pltpu.sync_copy(ych.at[pl.ds(nbase + stripe, CH)], ycc)
            pltpu.sync_copy(xch.at[pl.ds(nbase + stripe, CH)], xcc)

            def d_body(i, _d):
                lpos = i * 16 + iota
                f16 = plsc.load_gather(fdb, [lpos])
                c16 = plsc.load_gather(cdb, [lpos])
                plsc.store_scatter(fi0, [lpos], f16 * 4)
                plsc.store_scatter(fi1, [lpos], f16 * 4 + 1)
                plsc.store_scatter(fi2, [lpos], f16 * 4 + 2)
                plsc.store_scatter(fi3, [lpos], f16 * 4 + 3)
                plsc.store_scatter(cj1, [lpos], c16 * 4 + 1)
                plsc.store_scatter(cj2, [lpos], c16 * 4 + 2)
                plsc.store_scatter(cj3, [lpos], c16 * 4 + 3)
                return 0

            lax.fori_loop(0, NVEC, d_body, 0)
            pltpu.sync_copy(pxc, ftab.at[fi0])
            pltpu.sync_copy(pyc, ftab.at[fi1])
            pltpu.sync_copy(pzc, ftab.at[fi2])
            pltpu.sync_copy(pwc, ftab.at[fi3])
            pltpu.sync_copy(zcc, ctab.at[cj1])
            pltpu.sync_copy(ycc, ctab.at[cj2])
            pltpu.sync_copy(xcc, ctab.at[cj3])
            pltpu.sync_copy(onesb, otab.at[odb], add=True)
            plsc.subcore_barrier()

            fs = FD * 4 // NW
            pltpu.sync_copy(ftab.at[pl.ds(wid * fs, fs)],
                            feat_hbm.at[pl.ds(b * FD * 4 + wid * fs, fs)])
            cs = MAX_VOXELS * 4 // NW
            pltpu.sync_copy(
                ctab.at[pl.ds(wid * cs, cs)],
                coords_hbm.at[pl.ds(b * MAX_VOXELS * 4 + wid * cs, cs)])
            osz = MAX_VOXELS // NW
            pltpu.sync_copy(
                otab.at[pl.ds(wid * osz, osz)],
                occ_hbm.at[pl.ds(b * MAX_VOXELS + wid * osz, osz)])
            plsc.subcore_barrier()
            return 0

        lax.fori_loop(0, B // 2, do_batch, 0)

    return vox


_SC_VOX = _make_sc()


def kernel(points):
    xs = points[..., 0].reshape(B, ROWS, 128)
    ys = points[..., 1].reshape(B, ROWS, 128)
    zs = points[..., 2].reshape(B, ROWS, 128)
    keyp, zcp, ycp, xcp = _keys_stage(xs, ys, zs)
    kp = _fps_stage(xs, ys, zs)
    keys_flat = keyp.reshape(B * N)
    zc_flat = zcp.reshape(B * N)
    yc_flat = ycp.reshape(B * N)
    xc_flat = xcp.reshape(B * N)
    px = points[..., 0].reshape(B * N)
    py = points[..., 1].reshape(B * N)
    pz = points[..., 2].reshape(B * N)
    pw = points[..., 3].reshape(B * N)
    feat, coords, occ = _SC_VOX(keys_flat, zc_flat, yc_flat, xc_flat,
                                px, py, pz, pw)
    features = feat.reshape(B * MAX_VOXELS, MAX_OCC, 4)
    coordinates = coords.reshape(B * MAX_VOXELS, 4)
    occupancy = occ.reshape(B * MAX_VOXELS)
    return points, features, coordinates, occupancy, kp
